# baseline (speedup 1.0000x reference)
import functools

import jax
import jax.numpy as jnp
from jax.experimental import pallas as pl
from jax.experimental.pallas import tpu as pltpu


_BT = 4096

_K0 = 0
_K1 = 42
_KS2 = 0x1BD11BDA ^ _K0 ^ _K1
_ROT_A = (13, 15, 26, 6)
_ROT_B = (17, 29, 16, 24)


def _rotl(v, d):
    return (v << jnp.uint32(d)) | (v >> jnp.uint32(32 - d))


def _threefry_bits(cnt):
    ks0 = jnp.uint32(_K0)
    ks1 = jnp.uint32(_K1)
    ks2 = jnp.uint32(_KS2)
    x0 = jnp.zeros_like(cnt) + ks0
    x1 = cnt + ks1

    def four(x0, x1, rots):
        for r in rots:
            x0 = x0 + x1
            x1 = _rotl(x1, r)
            x1 = x1 ^ x0
        return x0, x1

    x0, x1 = four(x0, x1, _ROT_A)
    x0 = x0 + ks1
    x1 = x1 + ks2 + jnp.uint32(1)
    x0, x1 = four(x0, x1, _ROT_B)
    x0 = x0 + ks2
    x1 = x1 + ks0 + jnp.uint32(2)
    x0, x1 = four(x0, x1, _ROT_A)
    x0 = x0 + ks0
    x1 = x1 + ks1 + jnp.uint32(3)
    x0, x1 = four(x0, x1, _ROT_B)
    x0 = x0 + ks1
    x1 = x1 + ks2 + jnp.uint32(4)
    x0, x1 = four(x0, x1, _ROT_A)
    x0 = x0 + ks2
    x1 = x1 + ks0 + jnp.uint32(5)
    return x0 ^ x1


def _gumbel_block(t0, n_experts, bt):
    e_iota = jax.lax.broadcasted_iota(jnp.uint32, (n_experts, bt), 0)
    t_iota = jax.lax.broadcasted_iota(jnp.uint32, (n_experts, bt), 1)
    cnt = (jnp.uint32(t0) + t_iota) * jnp.uint32(n_experts) + e_iota
    bits = _threefry_bits(cnt)
    fbits = (bits >> jnp.uint32(9)) | jnp.uint32(0x3F800000)
    flo = jax.lax.bitcast_convert_type(fbits, jnp.float32) - jnp.float32(1.0)
    minval = jnp.float32(1e-20)
    maxval = jnp.float32(1.0)
    u = jnp.maximum(minval, flo * (maxval - minval) + minval)
    return -jnp.log(-jnp.log(u))


def _gate_kernel(xblk_ref, wt_ref, b_ref, o_ref, acc_ref):
    i = pl.program_id(0)
    j = pl.program_id(1)
    n_experts = o_ref.shape[0]

    scores = jax.lax.dot_general(
        xblk_ref[...], wt_ref[...], (((1,), (1,)), ((), ())),
        preferred_element_type=jnp.float32)

    @pl.when(j == 0)
    def _first():
        acc_ref[...] = scores.T

    @pl.when(j == 1)
    def _second():
        gumbels = _gumbel_block(i * _BT, n_experts, _BT)
        y_t = (acc_ref[...] + scores.T) + b_ref[...] + gumbels
        m = jnp.max(y_t, axis=0, keepdims=True)
        e_iota = jax.lax.broadcasted_iota(jnp.int32, y_t.shape, 0)
        sel = jnp.min(
            jnp.where(y_t == m, e_iota, n_experts), axis=0, keepdims=True)
        o_ref[...] = (e_iota == sel).astype(jnp.float32)


@functools.partial(jax.jit, static_argnames=())
def kernel(x, gate_weight, gate_bias):
    n_tokens, d_model = x.shape
    n_experts = gate_weight.shape[0]
    b2 = gate_bias.reshape(n_experts, 1)
    grid = (n_tokens // _BT, 2)
    dh = d_model // 2
    return pl.pallas_call(
        _gate_kernel,
        grid=grid,
        in_specs=[
            pl.BlockSpec((_BT, dh), lambda i, j: (i, j)),
            pl.BlockSpec((n_experts, dh), lambda i, j: (0, j)),
            pl.BlockSpec((n_experts, 1), lambda i, j: (0, 0)),
        ],
        out_specs=pl.BlockSpec((n_experts, _BT), lambda i, j: (0, i)),
        out_shape=jax.ShapeDtypeStruct((n_experts, n_tokens), x.dtype),
        scratch_shapes=[
            pltpu.VMEM((n_experts, _BT), jnp.float32),
        ],
        compiler_params=pltpu.CompilerParams(
            dimension_semantics=("parallel", "arbitrary")),
    )(x, gate_weight, b2).T


# speedup vs baseline: 1.1747x; 1.1747x over previous
import functools

import jax
import jax.numpy as jnp
from jax.experimental import pallas as pl
from jax.experimental.pallas import tpu as pltpu


_BT = 4096

_K0 = 0
_K1 = 42
_KS2 = 0x1BD11BDA ^ _K0 ^ _K1
_ROT_A = (13, 15, 26, 6)
_ROT_B = (17, 29, 16, 24)


def _rotl(v, d):
    return (v << jnp.uint32(d)) | (v >> jnp.uint32(32 - d))


def _threefry_bits(cnt):
    ks0 = jnp.uint32(_K0)
    ks1 = jnp.uint32(_K1)
    ks2 = jnp.uint32(_KS2)
    x0 = jnp.zeros_like(cnt) + ks0
    x1 = cnt + ks1

    def four(x0, x1, rots):
        for r in rots:
            x0 = x0 + x1
            x1 = _rotl(x1, r)
            x1 = x1 ^ x0
        return x0, x1

    x0, x1 = four(x0, x1, _ROT_A)
    x0 = x0 + ks1
    x1 = x1 + ks2 + jnp.uint32(1)
    x0, x1 = four(x0, x1, _ROT_B)
    x0 = x0 + ks2
    x1 = x1 + ks0 + jnp.uint32(2)
    x0, x1 = four(x0, x1, _ROT_A)
    x0 = x0 + ks0
    x1 = x1 + ks1 + jnp.uint32(3)
    x0, x1 = four(x0, x1, _ROT_B)
    x0 = x0 + ks1
    x1 = x1 + ks2 + jnp.uint32(4)
    x0, x1 = four(x0, x1, _ROT_A)
    x0 = x0 + ks2
    x1 = x1 + ks0 + jnp.uint32(5)
    return x0 ^ x1


def _gumbel_block(t0, n_experts, bt):
    e_iota = jax.lax.broadcasted_iota(jnp.uint32, (n_experts, bt), 0)
    t_iota = jax.lax.broadcasted_iota(jnp.uint32, (n_experts, bt), 1)
    cnt = (jnp.uint32(t0) + t_iota) * jnp.uint32(n_experts) + e_iota
    bits = _threefry_bits(cnt)
    fbits = (bits >> jnp.uint32(9)) | jnp.uint32(0x3F800000)
    flo = jax.lax.bitcast_convert_type(fbits, jnp.float32) - jnp.float32(1.0)
    minval = jnp.float32(1e-20)
    maxval = jnp.float32(1.0)
    u = jnp.maximum(minval, flo * (maxval - minval) + minval)
    return -jnp.log(-jnp.log(u))


def _gate_kernel(xblk_ref, wt_ref, b_ref, o_ref):
    i = pl.program_id(0)
    n_experts = o_ref.shape[0]

    gumbels = _gumbel_block(i * _BT, n_experts, _BT)
    xblk = xblk_ref[...]

    scores = jax.lax.dot_general(
        xblk, wt_ref[...], (((1,), (1,)), ((), ())),
        preferred_element_type=jnp.float32)
    y_t = scores.T + b_ref[...] + gumbels
    m = jnp.max(y_t, axis=0, keepdims=True)
    e_iota = jax.lax.broadcasted_iota(jnp.int32, y_t.shape, 0)
    sel = jnp.min(jnp.where(y_t == m, e_iota, n_experts), axis=0, keepdims=True)
    o_ref[...] = (e_iota == sel).astype(jnp.float32)


@functools.partial(jax.jit, static_argnames=())
def kernel(x, gate_weight, gate_bias):
    n_tokens, d_model = x.shape
    n_experts = gate_weight.shape[0]
    b2 = gate_bias.reshape(n_experts, 1)
    grid = (n_tokens // _BT,)
    return pl.pallas_call(
        _gate_kernel,
        grid=grid,
        in_specs=[
            pl.BlockSpec((_BT, d_model), lambda i: (i, 0)),
            pl.BlockSpec((n_experts, d_model), lambda i: (0, 0)),
            pl.BlockSpec((n_experts, 1), lambda i: (0, 0)),
        ],
        out_specs=pl.BlockSpec((n_experts, _BT), lambda i: (0, i)),
        out_shape=jax.ShapeDtypeStruct((n_experts, n_tokens), x.dtype),
        compiler_params=pltpu.CompilerParams(
            dimension_semantics=("parallel",)),
    )(x, gate_weight, b2).T
